# baseline (speedup 1.0000x reference)
import functools

import jax
import jax.numpy as jnp
from jax import lax
from jax.experimental import pallas as pl
from jax.experimental.pallas import tpu as pltpu
from jax.experimental.pallas import tpu_sc as plsc

_NUM_CORES = 2
_NUM_SUBCORES = 16
_NUM_WORKERS = _NUM_CORES * _NUM_SUBCORES

_BS_PER_CHUNK = 4
_NBUF = 4
_LANES = 16


def _gather_kernel(n_chunks, bpc, hist, hist_pad, ids_hbm, table_hbm, out_hbm,
                   idx_t, idx_v, rows_v, gsems, wsems):
  wid = lax.axis_index("s") * _NUM_CORES + lax.axis_index("c")
  bs_per_worker = n_chunks * bpc
  b_base = wid * bs_per_worker
  dim = table_hbm.shape[-1]
  n_trows = ids_hbm.shape[0]

  for t in range(n_trows):
    pltpu.sync_copy(ids_hbm.at[t, pl.ds(32 * wid, 32)],
                    idx_t.at[pl.ds(32 * t, 32)])

  n_vec = (hist_pad + _LANES - 1) // _LANES
  iota = jax.lax.iota(jnp.int32, _LANES)

  @pl.loop(0, bs_per_worker)
  def _(bl):
    c_local = bl >> 7
    l = bl & 127
    l_ix = jnp.zeros((_LANES,), jnp.int32) + l
    for m in range(n_vec):
      hh = iota + _LANES * m
      r_ix = ((hh >> 3) << 5) + (c_local << 3) + (hh & 7)
      if (m + 1) * _LANES > hist_pad:
        r_ix = jnp.where(hh < hist_pad, r_ix, 0)
      vals = plsc.load_gather(idx_t, [r_ix, l_ix])
      idx_v[bl, pl.ds(_LANES * m, _LANES)] = vals

  def start_gather(j, b):
    for k in range(bpc):
      pltpu.async_copy(table_hbm.at[idx_v.at[j * bpc + k, pl.ds(0, hist_pad)]],
                       rows_v.at[b, k], gsems[b])

  def wait_gather(j, b):
    for k in range(bpc):
      pltpu.make_async_copy(
          table_hbm.at[idx_v.at[j * bpc + k, pl.ds(0, hist_pad)]],
          rows_v.at[b, k], gsems[b]).wait()

  def out_slice(j, k):
    row0 = (b_base + j * bpc + k) * hist_pad
    return out_hbm.at[pl.ds(row0, hist), pl.ds(0, dim)]

  def start_write(j, b):
    for k in range(bpc):
      pltpu.async_copy(rows_v.at[b, k, pl.ds(0, hist)], out_slice(j, k),
                       wsems[b])

  def wait_write(j, b):
    for k in range(bpc):
      pltpu.make_async_copy(rows_v.at[b, k, pl.ds(0, hist)], out_slice(j, k),
                            wsems[b]).wait()

  for b in range(_NBUF):
    start_gather(b, b)

  n_groups = n_chunks // _NBUF

  @pl.loop(0, n_groups - 1)
  def _(g):
    first = g * _NBUF
    for b in range(_NBUF):
      wait_gather(first + b, b)
      start_write(first + b, b)
    for b in range(_NBUF):
      wait_write(first + b, b)
      start_gather(first + _NBUF + b, b)

  last = (n_groups - 1) * _NBUF
  for b in range(_NBUF):
    wait_gather(last + b, b)
    start_write(last + b, b)
  for b in range(_NBUF):
    wait_write(last + b, b)


def kernel(relation_ids, table):
  batch, hist = relation_ids.shape
  vocab, dim = table.shape
  hist_pad = (hist + 7) // 8 * 8
  dim_pad = 128
  assert batch % (_NUM_WORKERS * _BS_PER_CHUNK * _NBUF) == 0
  bs_per_worker = batch // _NUM_WORKERS
  n_chunks = bs_per_worker // _BS_PER_CHUNK
  n_trows = hist_pad // 8

  ids = jnp.pad(relation_ids.astype(jnp.int32).T, ((0, hist_pad - hist),
                                                   (0, 0)))
  ids = ids.reshape(n_trows, 8, batch // 128, 128).transpose(0, 2, 1, 3)
  ids = ids.reshape(n_trows, batch // 16, 128)

  mesh = plsc.VectorSubcoreMesh(core_axis_name="c", subcore_axis_name="s")
  grab = pl.kernel(
      functools.partial(_gather_kernel, n_chunks, _BS_PER_CHUNK, hist,
                        hist_pad),
      out_type=jax.ShapeDtypeStruct((batch * hist_pad, dim_pad), jnp.float32),
      mesh=mesh,
      scratch_types=[
          pltpu.VMEM((n_trows * 32, 128), jnp.int32),
          pltpu.VMEM((bs_per_worker, 64), jnp.int32),
          pltpu.VMEM((_NBUF, _BS_PER_CHUNK, hist_pad, dim), jnp.float32),
          [pltpu.SemaphoreType.DMA] * _NBUF,
          [pltpu.SemaphoreType.DMA] * _NBUF,
      ],
      compiler_params=pltpu.CompilerParams(use_tc_tiling_on_sc=False,
                                           needs_layout_passes=False),
  )
  out = grab(ids, table)
  return out.reshape(batch, hist_pad, dim_pad)[:, :hist, :dim]


# speedup vs baseline: 6.2251x; 6.2251x over previous
import functools

import jax
import jax.numpy as jnp
from jax import lax
from jax.experimental import pallas as pl
from jax.experimental.pallas import tpu as pltpu
from jax.experimental.pallas import tpu_sc as plsc

_NUM_CORES = 2
_NUM_SUBCORES = 16
_NUM_WORKERS = _NUM_CORES * _NUM_SUBCORES

_BS_PER_CHUNK = 4
_NBUF = 4


def _gather_kernel(n_chunks, bpc, hist, hist_pad, ids_hbm, table_hbm, out_hbm,
                   idx_v, rows_v, gsems, wsems):
  wid = lax.axis_index("s") * _NUM_CORES + lax.axis_index("c")
  bs_per_worker = n_chunks * bpc
  b_base = wid * bs_per_worker
  dim = table_hbm.shape[-1]

  pltpu.sync_copy(ids_hbm.at[pl.ds(b_base, bs_per_worker)], idx_v)

  def start_gather(j, b):
    for k in range(bpc):
      pltpu.async_copy(table_hbm.at[idx_v.at[j * bpc + k]],
                       rows_v.at[b, k], gsems[b])

  def wait_gather(j, b):
    for k in range(bpc):
      pltpu.make_async_copy(table_hbm.at[idx_v.at[j * bpc + k]],
                            rows_v.at[b, k], gsems[b]).wait()

  def out_slice(j, k):
    row0 = (b_base + j * bpc + k) * hist_pad
    return out_hbm.at[pl.ds(row0, hist), pl.ds(0, dim)]

  def start_write(j, b):
    for k in range(bpc):
      pltpu.async_copy(rows_v.at[b, k], out_slice(j, k), wsems[b])

  def wait_write(j, b):
    for k in range(bpc):
      pltpu.make_async_copy(rows_v.at[b, k], out_slice(j, k), wsems[b]).wait()

  for b in range(_NBUF):
    start_gather(b, b)

  n_groups = n_chunks // _NBUF

  @pl.loop(0, n_groups - 1)
  def _(g):
    first = g * _NBUF
    for b in range(_NBUF):
      wait_gather(first + b, b)
      start_write(first + b, b)
    for b in range(_NBUF):
      wait_write(first + b, b)
      start_gather(first + _NBUF + b, b)

  last = (n_groups - 1) * _NBUF
  for b in range(_NBUF):
    wait_gather(last + b, b)
    start_write(last + b, b)
  for b in range(_NBUF):
    wait_write(last + b, b)


def kernel(relation_ids, table):
  batch, hist = relation_ids.shape
  vocab, dim = table.shape
  hist_pad = (hist + 7) // 8 * 8
  dim_pad = 128
  assert batch % (_NUM_WORKERS * _BS_PER_CHUNK * _NBUF) == 0
  bs_per_worker = batch // _NUM_WORKERS
  n_chunks = bs_per_worker // _BS_PER_CHUNK

  ids = relation_ids.astype(jnp.int32)

  mesh = plsc.VectorSubcoreMesh(core_axis_name="c", subcore_axis_name="s")
  grab = pl.kernel(
      functools.partial(_gather_kernel, n_chunks, _BS_PER_CHUNK, hist,
                        hist_pad),
      out_type=jax.ShapeDtypeStruct((batch * hist_pad, dim_pad), jnp.float32),
      mesh=mesh,
      scratch_types=[
          pltpu.VMEM((bs_per_worker, hist), jnp.int32),
          pltpu.VMEM((_NBUF, _BS_PER_CHUNK, hist, dim), jnp.float32),
          [pltpu.SemaphoreType.DMA] * _NBUF,
          [pltpu.SemaphoreType.DMA] * _NBUF,
      ],
      compiler_params=pltpu.CompilerParams(use_tc_tiling_on_sc=False),
  )
  out = grab(ids, table)
  return out.reshape(batch, hist_pad, dim_pad)[:, :hist, :dim]
